# baseline (device time: 12733 ns/iter reference)
import jax
import jax.numpy as jnp
from jax import lax
from jax.experimental import pallas as pl
from jax.experimental.pallas import tpu as pltpu

N_DEV = 4
T = 64
FIX_STEPS = T

_BATCH_DOT = (((2,), (1,)), ((0,), (0,)))


def kernel(x, A, B, C):
    Bb, S, D = x.shape
    N = A.shape[-1]
    AT = A.T
    nblk = S // T

    def body(x_ref, at_ref, b_ref, c_ref, out_ref,
             hend_ref, carry_ref, send_sem, recv_sem):
        my = lax.axis_index("i")
        left = (my - 1) % N_DEV
        right = (my + 1) % N_DEV

        barrier = pltpu.get_barrier_semaphore()
        for nbr in (left, right):
            pl.semaphore_signal(
                barrier, inc=1,
                device_id=(nbr,), device_id_type=pl.DeviceIdType.MESH,
            )
        pl.semaphore_wait(barrier, 2)

        dAT = jnp.exp(at_ref[:, :])[None].astype(jnp.bfloat16)

        c_all = c_ref[...].astype(jnp.bfloat16)
        ctile = jnp.concatenate([c_all] * T, axis=2)
        lane = lax.broadcasted_iota(jnp.int32, (Bb, S, T * N), 2)
        trow = lax.broadcasted_iota(jnp.int32, (Bb, S, T * N), 1)
        cbd_all = jnp.where((lane // N) == (trow % T), ctile,
                            jnp.bfloat16(0))

        rown = lax.broadcasted_iota(jnp.int32, (T * N, N), 0) % N
        coln = lax.broadcasted_iota(jnp.int32, (T * N, N), 1)
        bmask = (rown == coln)[None]
        ones_nd = jnp.ones((Bb, N, D), jnp.bfloat16)

        h = jnp.zeros((Bb, N, D), jnp.bfloat16)
        for k in range(nblk):
            xblk = x_ref[:, k * T:(k + 1) * T, :].astype(jnp.bfloat16)
            bblk = b_ref[:, k * T:(k + 1) * T, :].astype(jnp.bfloat16)
            cbd = cbd_all[:, k * T:(k + 1) * T, :]
            bup = jnp.broadcast_to(
                bblk[:, :, None, :], (Bb, T, N, N)).reshape(Bb, T * N, N)
            bsel = jnp.where(bmask, bup, jnp.bfloat16(0))
            bsplat = lax.dot_general(
                bsel, ones_nd, _BATCH_DOT, preferred_element_type=jnp.float32,
            ).astype(jnp.bfloat16)
            hs = []
            for j in range(T):
                h = (h * dAT
                     + bsplat[:, j * N:(j + 1) * N, :] * xblk[:, j:j + 1, :])
                hs.append(h)
            hstack = jnp.concatenate(hs, axis=1)
            out_ref[:, k * T:(k + 1) * T, :] = lax.dot_general(
                cbd, hstack, _BATCH_DOT, preferred_element_type=jnp.float32)
        hend_ref[...] = h

        rdma = pltpu.make_async_remote_copy(
            src_ref=hend_ref,
            dst_ref=carry_ref,
            send_sem=send_sem,
            recv_sem=recv_sem,
            device_id=(right,),
            device_id_type=pl.DeviceIdType.MESH,
        )
        rdma.start()
        rdma.wait()

        @pl.when(my != 0)
        def _fixup():
            g = carry_ref[...]
            for k in range(FIX_STEPS // T):
                cbd = cbd_all[:, k * T:(k + 1) * T, :]
                gs = []
                for j in range(T):
                    g = g * dAT
                    gs.append(g)
                gstack = jnp.concatenate(gs, axis=1)
                dy = lax.dot_general(
                    cbd, gstack, _BATCH_DOT,
                    preferred_element_type=jnp.float32)
                out_ref[:, k * T:(k + 1) * T, :] = (
                    out_ref[:, k * T:(k + 1) * T, :] + dy)

    return pl.pallas_call(
        body,
        out_shape=jax.ShapeDtypeStruct((Bb, S, D), jnp.float32),
        in_specs=[pl.BlockSpec(memory_space=pltpu.VMEM)] * 4,
        out_specs=pl.BlockSpec(memory_space=pltpu.VMEM),
        scratch_shapes=[
            pltpu.VMEM((Bb, N, D), jnp.bfloat16),
            pltpu.VMEM((Bb, N, D), jnp.bfloat16),
            pltpu.SemaphoreType.DMA,
            pltpu.SemaphoreType.DMA,
        ],
        compiler_params=pltpu.CompilerParams(collective_id=0),
    )(x, AT, B, C)


# device time: 11214 ns/iter; 1.1355x vs baseline; 1.1355x over previous
import jax
import jax.numpy as jnp
from jax import lax
from jax.experimental import pallas as pl
from jax.experimental.pallas import tpu as pltpu

N_DEV = 4
T = 32
FIX_STEPS = 32

_BATCH_DOT = (((2,), (1,)), ((0,), (0,)))


def kernel(x, A, B, C):
    Bb, S, D = x.shape
    N = A.shape[-1]
    AT = A.T
    nblk = S // T

    def body(x_ref, at_ref, b_ref, c_ref, out_ref,
             hend_ref, carry_ref, send_sem, recv_sem):
        my = lax.axis_index("i")
        left = (my - 1) % N_DEV
        right = (my + 1) % N_DEV

        barrier = pltpu.get_barrier_semaphore()
        pl.semaphore_signal(
            barrier, inc=1,
            device_id=(left,), device_id_type=pl.DeviceIdType.MESH,
        )
        pl.semaphore_wait(barrier, 1)

        dAT = jnp.exp(at_ref[:, :])[None].astype(jnp.bfloat16)

        c_all = c_ref[...].astype(jnp.bfloat16)
        ctile = jnp.concatenate([c_all] * T, axis=2)
        lane = lax.broadcasted_iota(jnp.int32, (Bb, S, T * N), 2)
        trow = lax.broadcasted_iota(jnp.int32, (Bb, S, T * N), 1)
        cbd_all = jnp.where((lane // N) == (trow % T), ctile,
                            jnp.bfloat16(0))

        rown = lax.broadcasted_iota(jnp.int32, (T * N, N), 0) % N
        coln = lax.broadcasted_iota(jnp.int32, (T * N, N), 1)
        bmask = (rown == coln)[None]
        ones_nd = jnp.ones((Bb, N, D), jnp.bfloat16)

        rdma = pltpu.make_async_remote_copy(
            src_ref=hend_ref,
            dst_ref=carry_ref,
            send_sem=send_sem,
            recv_sem=recv_sem,
            device_id=(right,),
            device_id_type=pl.DeviceIdType.MESH,
        )

        def chain(k, h):
            xblk = x_ref[:, k * T:(k + 1) * T, :].astype(jnp.bfloat16)
            bblk = b_ref[:, k * T:(k + 1) * T, :].astype(jnp.bfloat16)
            bup = jnp.broadcast_to(
                bblk[:, :, None, :], (Bb, T, N, N)).reshape(Bb, T * N, N)
            bsel = jnp.where(bmask, bup, jnp.bfloat16(0))
            bsplat = lax.dot_general(
                bsel, ones_nd, _BATCH_DOT, preferred_element_type=jnp.float32,
            ).astype(jnp.bfloat16)
            hs = []
            for j in range(T):
                h = (h * dAT
                     + bsplat[:, j * N:(j + 1) * N, :] * xblk[:, j:j + 1, :])
                hs.append(h)
            return h, jnp.concatenate(hs, axis=1)

        def store_y(k, hstack):
            cbd = cbd_all[:, k * T:(k + 1) * T, :]
            out_ref[:, k * T:(k + 1) * T, :] = lax.dot_general(
                cbd, hstack, _BATCH_DOT, preferred_element_type=jnp.float32)

        h = jnp.zeros((Bb, N, D), jnp.bfloat16)
        for k in range(nblk - 1):
            h, hstack = chain(k, h)
            store_y(k, hstack)
        h, hstack = chain(nblk - 1, h)
        hend_ref[...] = h
        rdma.start()
        store_y(nblk - 1, hstack)
        rdma.wait()

        @pl.when(my != 0)
        def _fixup():
            g = carry_ref[...]
            for k in range(FIX_STEPS // T):
                cbd = cbd_all[:, k * T:(k + 1) * T, :]
                gs = []
                for j in range(T):
                    g = g * dAT
                    gs.append(g)
                gstack = jnp.concatenate(gs, axis=1)
                dy = lax.dot_general(
                    cbd, gstack, _BATCH_DOT,
                    preferred_element_type=jnp.float32)
                out_ref[:, k * T:(k + 1) * T, :] = (
                    out_ref[:, k * T:(k + 1) * T, :] + dy)

    return pl.pallas_call(
        body,
        out_shape=jax.ShapeDtypeStruct((Bb, S, D), jnp.float32),
        in_specs=[pl.BlockSpec(memory_space=pltpu.VMEM)] * 4,
        out_specs=pl.BlockSpec(memory_space=pltpu.VMEM),
        scratch_shapes=[
            pltpu.VMEM((Bb, N, D), jnp.bfloat16),
            pltpu.VMEM((Bb, N, D), jnp.bfloat16),
            pltpu.SemaphoreType.DMA,
            pltpu.SemaphoreType.DMA,
        ],
        compiler_params=pltpu.CompilerParams(collective_id=0),
    )(x, AT, B, C)


# device time: 10999 ns/iter; 1.1577x vs baseline; 1.0195x over previous
import jax
import jax.numpy as jnp
from jax import lax
from jax.experimental import pallas as pl
from jax.experimental.pallas import tpu as pltpu

N_DEV = 4
T = 32
TI = 8
FIX_STEPS = 32

_BATCH_DOT = (((2,), (1,)), ((0,), (0,)))


def kernel(x, A, B, C):
    Bb, S, D = x.shape
    N = A.shape[-1]
    AT = A.T
    nblk = S // T

    def body(x_ref, at_ref, b_ref, c_ref, out_ref,
             hend_ref, carry_ref, send_sem, recv_sem):
        my = lax.axis_index("i")
        left = (my - 1) % N_DEV
        right = (my + 1) % N_DEV

        barrier = pltpu.get_barrier_semaphore()
        pl.semaphore_signal(
            barrier, inc=1,
            device_id=(left,), device_id_type=pl.DeviceIdType.MESH,
        )

        dAT = jnp.exp(at_ref[:, :])[None].astype(jnp.bfloat16)

        c_all = c_ref[...].astype(jnp.bfloat16)
        ctile = jnp.concatenate([c_all] * TI, axis=2)
        lane = lax.broadcasted_iota(jnp.int32, (Bb, S, TI * N), 2)
        trow = lax.broadcasted_iota(jnp.int32, (Bb, S, TI * N), 1)
        cbd_all = jnp.where((lane // N) == (trow % TI), ctile,
                            jnp.bfloat16(0))

        rown = lax.broadcasted_iota(jnp.int32, (T * N, N), 0) % N
        coln = lax.broadcasted_iota(jnp.int32, (T * N, N), 1)
        bmask = (rown == coln)[None]
        ones_nd = jnp.ones((Bb, N, D), jnp.bfloat16)

        rdma = pltpu.make_async_remote_copy(
            src_ref=hend_ref,
            dst_ref=carry_ref,
            send_sem=send_sem,
            recv_sem=recv_sem,
            device_id=(right,),
            device_id_type=pl.DeviceIdType.MESH,
        )

        def chain(k, h):
            xblk = x_ref[:, k * T:(k + 1) * T, :].astype(jnp.bfloat16)
            bblk = b_ref[:, k * T:(k + 1) * T, :].astype(jnp.bfloat16)
            bup = jnp.broadcast_to(
                bblk[:, :, None, :], (Bb, T, N, N)).reshape(Bb, T * N, N)
            bsel = jnp.where(bmask, bup, jnp.bfloat16(0))
            bsplat = lax.dot_general(
                bsel, ones_nd, _BATCH_DOT, preferred_element_type=jnp.float32,
            ).astype(jnp.bfloat16)
            hs = []
            for j in range(T):
                h = (h * dAT
                     + bsplat[:, j * N:(j + 1) * N, :] * xblk[:, j:j + 1, :])
                hs.append(h)
            return h, hs

        def store_y(k, hs):
            for m in range(T // TI):
                t0 = k * T + m * TI
                cbd = cbd_all[:, t0:t0 + TI, :]
                hstack = jnp.concatenate(hs[m * TI:(m + 1) * TI], axis=1)
                out_ref[:, t0:t0 + TI, :] = lax.dot_general(
                    cbd, hstack, _BATCH_DOT,
                    preferred_element_type=jnp.float32)

        h = jnp.zeros((Bb, N, D), jnp.bfloat16)
        for k in range(nblk - 1):
            h, hs = chain(k, h)
            store_y(k, hs)
        h, hs = chain(nblk - 1, h)
        hend_ref[...] = h
        pl.semaphore_wait(barrier, 1)
        rdma.start()
        store_y(nblk - 1, hs)

        ps = []
        p = dAT
        for _ in range(FIX_STEPS):
            ps.append(p)
            p = p * dAT
        pstacks = [
            jnp.concatenate(ps[k * TI:(k + 1) * TI], axis=1)
            for k in range(FIX_STEPS // TI)
        ]

        rdma.wait_recv()

        @pl.when(my != 0)
        def _fixup():
            ctile_carry = jnp.concatenate(
                [carry_ref[...]] * TI, axis=1)
            for k in range(FIX_STEPS // TI):
                cbd = cbd_all[:, k * TI:(k + 1) * TI, :]
                gstack = pstacks[k] * ctile_carry
                dy = lax.dot_general(
                    cbd, gstack, _BATCH_DOT,
                    preferred_element_type=jnp.float32)
                out_ref[:, k * TI:(k + 1) * TI, :] = (
                    out_ref[:, k * TI:(k + 1) * TI, :] + dy)

        rdma.wait_send()

    return pl.pallas_call(
        body,
        out_shape=jax.ShapeDtypeStruct((Bb, S, D), jnp.float32),
        in_specs=[pl.BlockSpec(memory_space=pltpu.VMEM)] * 4,
        out_specs=pl.BlockSpec(memory_space=pltpu.VMEM),
        scratch_shapes=[
            pltpu.VMEM((Bb, N, D), jnp.bfloat16),
            pltpu.VMEM((Bb, N, D), jnp.bfloat16),
            pltpu.SemaphoreType.DMA,
            pltpu.SemaphoreType.DMA,
        ],
        compiler_params=pltpu.CompilerParams(collective_id=0),
    )(x, AT, B, C)
